# baseline (speedup 1.0000x reference)
import functools

import jax
import jax.numpy as jnp
from jax import lax
from jax.experimental import pallas as pl
from jax.experimental.pallas import tpu as pltpu
from jax.experimental.pallas import tpu_sc as plsc

_NW = 32
_RCH = 64
_CHS = 128


def _sc_body(td_ref, mask_ref, ll_ref, out_ref,
             buf0, buf1, tdb, mkb, wb, accv, sems, *, b_dim, s_dim, nk, inv_n):
    rows_per_w = (b_dim * s_dim) // _NW
    sh = s_dim // (_NW // b_dim)
    nchunk = rows_per_w // _RCH

    wid = lax.axis_index("s") * 2 + lax.axis_index("c")
    b = wid // (_NW // b_dim)
    s0 = (wid % (_NW // b_dim)) * sh

    bufs = (buf0, buf1)

    def cpy(c, slot):
        return pltpu.make_async_copy(
            ll_ref.at[b, pl.ds(s0 + c * _RCH, _RCH)], bufs[slot], sems.at[slot])

    pltpu.sync_copy(td_ref.at[b, pl.ds(s0, sh)], tdb)
    pltpu.sync_copy(mask_ref.at[b, pl.ds(s0, sh)], mkb)
    cpy(0, 0).start()
    for m in range(rows_per_w // 16):
        sl = pl.ds(m * 16, 16)
        wb[sl] = tdb[sl] * mkb[sl] * inv_n

    iota = lax.iota(jnp.int32, 16)
    total = jnp.zeros((16,), jnp.float32)
    for c in range(nchunk):
        slot = c & 1
        cpy(c, slot).wait()
        if c + 1 < nchunk:
            cpy(c + 1, 1 - slot).start()
        buf = bufs[slot]
        for g in range(_RCH // 16):
            idx_r = iota + (g * 16)

            def jblk(jj, acc, idx_r=idx_r, buf=buf):
                for u in range(16):
                    col = jj * 16 + u
                    acc = acc + plsc.load_gather(
                        buf, [idx_r, jnp.full((16,), col, jnp.int32)])
                return acc

            rowsums = lax.fori_loop(0, nk // 16, jblk, jnp.zeros((16,), jnp.float32))
            total = total + rowsums * wb[pl.ds(c * _RCH + g * 16, 16)]

    accv[...] = total
    pltpu.sync_copy(accv, out_ref.at[wid])


def _tc_body(mask_ref, lae_ref, ltm_ref, e2_ref, ev_ref, cnt_ref, acc_ev, *, k):
    i = pl.program_id(0)
    nsteps = pl.num_programs(0)

    @pl.when(i == 0)
    def _init():
        acc_ev[...] = jnp.zeros_like(acc_ev)
        cnt_ref[0, 0] = jnp.float32(0.0)

    mch = mask_ref[...]
    x = lae_ref[...] * ltm_ref[...]
    ev_l = jnp.dot(x, e2_ref[...], preferred_element_type=jnp.float32)
    acc_ev[...] += jnp.log(jnp.where(mch > 0, ev_l, 1.0))
    cnt_ref[0, 0] += jnp.sum(mch)

    @pl.when(i == nsteps - 1)
    def _fini():
        ev_ref[0, 0] = jnp.sum(acc_ev[...])


def kernel(time_delta_seq, lambda_at_event, lambdas_loss_samples, seq_mask, lambda_type_mask):
    B, S, N, K = lambdas_loss_samples.shape
    NK = N * K
    ll = lambdas_loss_samples.reshape(B, S, NK)
    laef = lambda_at_event.reshape(B, S * K)
    ltmf = lambda_type_mask.reshape(B, S * K)
    maskf = seq_mask.astype(jnp.float32)

    sh = S // (_NW // B)
    mesh = plsc.VectorSubcoreMesh(core_axis_name="c", subcore_axis_name="s")
    sc_fn = functools.partial(_sc_body, b_dim=B, s_dim=S, nk=NK, inv_n=1.0 / N)
    ne_parts = pl.kernel(
        sc_fn,
        out_type=jax.ShapeDtypeStruct((_NW, 16), jnp.float32),
        mesh=mesh,
        compiler_params=pltpu.CompilerParams(needs_layout_passes=False),
        scratch_types=[
            pltpu.VMEM((_RCH, NK), jnp.float32),
            pltpu.VMEM((_RCH, NK), jnp.float32),
            pltpu.VMEM((sh,), jnp.float32),
            pltpu.VMEM((sh,), jnp.float32),
            pltpu.VMEM((sh,), jnp.float32),
            pltpu.VMEM((16,), jnp.float32),
            pltpu.SemaphoreType.DMA((2,)),
        ],
    )(time_delta_seq, maskf, ll)

    e2 = jnp.kron(jnp.eye(_CHS, dtype=jnp.float32), jnp.ones((K, 1), jnp.float32))
    tc_fn = functools.partial(_tc_body, k=K)
    ev, cnt = pl.pallas_call(
        tc_fn,
        grid=(S // _CHS,),
        in_specs=[
            pl.BlockSpec((B, _CHS), lambda i: (0, i)),
            pl.BlockSpec((B, _CHS * K), lambda i: (0, i)),
            pl.BlockSpec((B, _CHS * K), lambda i: (0, i)),
            pl.BlockSpec((_CHS * K, _CHS), lambda i: (0, 0)),
        ],
        out_specs=[
            pl.BlockSpec(memory_space=pltpu.SMEM),
            pl.BlockSpec(memory_space=pltpu.SMEM),
        ],
        out_shape=[
            jax.ShapeDtypeStruct((1, 1), jnp.float32),
            jax.ShapeDtypeStruct((1, 1), jnp.float32),
        ],
        scratch_shapes=[
            pltpu.VMEM((B, _CHS), jnp.float32),
        ],
    )(maskf, laef, ltmf, e2)

    return (ev[0, 0], jnp.sum(ne_parts), cnt[0, 0].astype(jnp.int32))


# speedup vs baseline: 2.1690x; 2.1690x over previous
import functools

import jax
import jax.numpy as jnp
from jax import lax
from jax.experimental import pallas as pl
from jax.experimental.pallas import tpu as pltpu
from jax.experimental.pallas import tpu_sc as plsc

_NW = 32
_RCH = 64
_CHS = 128


def _sc_body(td_ref, mask_ref, ll_ref, out_ref,
             buf0, buf1, tdb, mkb, wb, accv, sems, *, b_dim, s_dim, nk, inv_n):
    rows_per_w = (b_dim * s_dim) // _NW
    sh = s_dim // (_NW // b_dim)
    nchunk = rows_per_w // _RCH

    wid = lax.axis_index("s") * 2 + lax.axis_index("c")
    b = wid // (_NW // b_dim)
    s0 = (wid % (_NW // b_dim)) * sh

    bufs = (buf0, buf1)

    def cpy(c, slot):
        return pltpu.make_async_copy(
            ll_ref.at[b, pl.ds(s0 + c * _RCH, _RCH)], bufs[slot], sems.at[slot])

    pltpu.sync_copy(td_ref.at[b, pl.ds(s0, sh)], tdb)
    pltpu.sync_copy(mask_ref.at[b, pl.ds(s0, sh)], mkb)
    cpy(0, 0).start()
    for m in range(rows_per_w // 16):
        sl = pl.ds(m * 16, 16)
        wb[sl] = tdb[sl] * mkb[sl] * inv_n

    nacc = 4
    total = jnp.zeros((16,), jnp.float32)
    for c in range(nchunk):
        slot = c & 1
        cpy(c, slot).wait()
        if c + 1 < nchunk:
            cpy(c + 1, 1 - slot).start()
        buf = bufs[slot]

        def rbody(r, tot, c=c, buf=buf):
            wsplat = plsc.load_gather(
                wb, [jnp.full((16,), c * _RCH + r, jnp.int32)])
            accs = [jnp.zeros((16,), jnp.float32) for _ in range(nacc)]
            for j in range(nk // 16):
                accs[j % nacc] = accs[j % nacc] + buf[r, pl.ds(j * 16, 16)]
            rowsum = (accs[0] + accs[1]) + (accs[2] + accs[3])
            return tot + rowsum * wsplat

        total = lax.fori_loop(0, _RCH, rbody, total)

    accv[...] = total
    pltpu.sync_copy(accv, out_ref.at[wid])


def _tc_body(mask_ref, lae_ref, ltm_ref, e2_ref, ev_ref, cnt_ref, acc_ev, *, k):
    i = pl.program_id(0)
    nsteps = pl.num_programs(0)

    @pl.when(i == 0)
    def _init():
        acc_ev[...] = jnp.zeros_like(acc_ev)
        cnt_ref[0, 0] = jnp.float32(0.0)

    mch = mask_ref[...]
    x = lae_ref[...] * ltm_ref[...]
    ev_l = jnp.dot(x, e2_ref[...], preferred_element_type=jnp.float32)
    acc_ev[...] += jnp.log(jnp.where(mch > 0, ev_l, 1.0))
    cnt_ref[0, 0] += jnp.sum(mch)

    @pl.when(i == nsteps - 1)
    def _fini():
        ev_ref[0, 0] = jnp.sum(acc_ev[...])


def kernel(time_delta_seq, lambda_at_event, lambdas_loss_samples, seq_mask, lambda_type_mask):
    B, S, N, K = lambdas_loss_samples.shape
    NK = N * K
    ll = lambdas_loss_samples.reshape(B, S, NK)
    laef = lambda_at_event.reshape(B, S * K)
    ltmf = lambda_type_mask.reshape(B, S * K)
    maskf = seq_mask.astype(jnp.float32)

    sh = S // (_NW // B)
    mesh = plsc.VectorSubcoreMesh(core_axis_name="c", subcore_axis_name="s")
    sc_fn = functools.partial(_sc_body, b_dim=B, s_dim=S, nk=NK, inv_n=1.0 / N)
    ne_parts = pl.kernel(
        sc_fn,
        out_type=jax.ShapeDtypeStruct((_NW, 16), jnp.float32),
        mesh=mesh,
        compiler_params=pltpu.CompilerParams(needs_layout_passes=False),
        scratch_types=[
            pltpu.VMEM((_RCH, NK), jnp.float32),
            pltpu.VMEM((_RCH, NK), jnp.float32),
            pltpu.VMEM((sh,), jnp.float32),
            pltpu.VMEM((sh,), jnp.float32),
            pltpu.VMEM((sh,), jnp.float32),
            pltpu.VMEM((16,), jnp.float32),
            pltpu.SemaphoreType.DMA((2,)),
        ],
    )(time_delta_seq, maskf, ll)

    e2 = jnp.kron(jnp.eye(_CHS, dtype=jnp.float32), jnp.ones((K, 1), jnp.float32))
    tc_fn = functools.partial(_tc_body, k=K)
    ev, cnt = pl.pallas_call(
        tc_fn,
        grid=(S // _CHS,),
        in_specs=[
            pl.BlockSpec((B, _CHS), lambda i: (0, i)),
            pl.BlockSpec((B, _CHS * K), lambda i: (0, i)),
            pl.BlockSpec((B, _CHS * K), lambda i: (0, i)),
            pl.BlockSpec((_CHS * K, _CHS), lambda i: (0, 0)),
        ],
        out_specs=[
            pl.BlockSpec(memory_space=pltpu.SMEM),
            pl.BlockSpec(memory_space=pltpu.SMEM),
        ],
        out_shape=[
            jax.ShapeDtypeStruct((1, 1), jnp.float32),
            jax.ShapeDtypeStruct((1, 1), jnp.float32),
        ],
        scratch_shapes=[
            pltpu.VMEM((B, _CHS), jnp.float32),
        ],
    )(maskf, laef, ltmf, e2)

    return (ev[0, 0], jnp.sum(ne_parts), cnt[0, 0].astype(jnp.int32))


# speedup vs baseline: 2.2864x; 1.0541x over previous
import functools

import jax
import jax.numpy as jnp
from jax import lax
from jax.experimental import pallas as pl
from jax.experimental.pallas import tpu as pltpu

_CHS = 128


def _body(td_ref, mask_ref, lae_ref, ltm_ref, e2_ref, ll_ref,
          ev_ref, ne_ref, cnt_ref, acc_ne, acc_ev, *, inv_n):
    i = pl.program_id(0)
    nsteps = pl.num_programs(0)

    @pl.when(i == 0)
    def _init():
        acc_ne[...] = jnp.zeros_like(acc_ne)
        acc_ev[...] = jnp.zeros_like(acc_ev)
        cnt_ref[0, 0] = jnp.float32(0.0)

    mch = mask_ref[...]
    w = td_ref[...] * mch * inv_n
    acc_ne[...] += lax.dot_general(
        w, ll_ref[...],
        dimension_numbers=(((1,), (1,)), ((0,), (0,))),
        preferred_element_type=jnp.float32,
    )

    x = lae_ref[...] * ltm_ref[...]
    ev_l = jnp.dot(x, e2_ref[...], preferred_element_type=jnp.float32)
    acc_ev[...] += jnp.log(jnp.where(mch > 0, ev_l, 1.0))
    cnt_ref[0, 0] += jnp.sum(mch)

    @pl.when(i == nsteps - 1)
    def _fini():
        ne_ref[0, 0] = jnp.sum(acc_ne[...])
        ev_ref[0, 0] = jnp.sum(acc_ev[...])


def kernel(time_delta_seq, lambda_at_event, lambdas_loss_samples, seq_mask, lambda_type_mask):
    B, S, N, K = lambdas_loss_samples.shape
    NK = N * K
    ll = lambdas_loss_samples.reshape(B, S, NK)
    laef = lambda_at_event.reshape(B, S * K)
    ltmf = lambda_type_mask.reshape(B, S * K)
    maskf = seq_mask.astype(jnp.float32)
    e2 = jnp.kron(jnp.eye(_CHS, dtype=jnp.float32), jnp.ones((K, 1), jnp.float32))

    body = functools.partial(_body, inv_n=1.0 / N)
    ev, ne, cnt = pl.pallas_call(
        body,
        grid=(S // _CHS,),
        in_specs=[
            pl.BlockSpec((B, _CHS), lambda i: (0, i)),
            pl.BlockSpec((B, _CHS), lambda i: (0, i)),
            pl.BlockSpec((B, _CHS * K), lambda i: (0, i)),
            pl.BlockSpec((B, _CHS * K), lambda i: (0, i)),
            pl.BlockSpec((_CHS * K, _CHS), lambda i: (0, 0)),
            pl.BlockSpec((B, _CHS, NK), lambda i: (0, i, 0)),
        ],
        out_specs=[
            pl.BlockSpec(memory_space=pltpu.SMEM),
            pl.BlockSpec(memory_space=pltpu.SMEM),
            pl.BlockSpec(memory_space=pltpu.SMEM),
        ],
        out_shape=[
            jax.ShapeDtypeStruct((1, 1), jnp.float32),
            jax.ShapeDtypeStruct((1, 1), jnp.float32),
            jax.ShapeDtypeStruct((1, 1), jnp.float32),
        ],
        scratch_shapes=[
            pltpu.VMEM((B, NK), jnp.float32),
            pltpu.VMEM((B, _CHS), jnp.float32),
        ],
    )(time_delta_seq, maskf, laef, ltmf, e2, ll)

    return (ev[0, 0], ne[0, 0], cnt[0, 0].astype(jnp.int32))
